# Initial kernel scaffold; baseline (speedup 1.0000x reference)
import jax
import jax.numpy as jnp
from jax.experimental import pallas as pl


def kernel(x_0, x_1, edges, edge_types, edge_attr, Qw, Qb, Kw, Kb, Vw, Vb, Ww, Wb, w_ih, w_hh, b_ih, b_hh):
    raise NotImplementedError("write your pallas kernel here")


# speedup vs baseline: 4.6758x; 4.6758x over previous
import dataclasses
import functools
import math

import jax
import jax.numpy as jnp
from jax import lax
from jax.experimental import pallas as pl
from jax.experimental.pallas import tpu as pltpu
from jax.experimental.pallas import tpu_sc as plsc

N = 10000
E = 320000
HID = 128

NC = 2
NS = 16
NW = NC * NS
PER_W = E // NW
CH = 80
NCHUNK = PER_W // CH
N_PAD = 10240
RSUB = N_PAD // NS

_f32 = jnp.float32
_mesh = plsc.VectorSubcoreMesh(core_axis_name="c", subcore_axis_name="s")

_sc_params = pltpu.CompilerParams()
if "needs_layout_passes" in pltpu.CompilerParams.__dataclass_fields__:
    _sc_params = dataclasses.replace(_sc_params, needs_layout_passes=False)


def _node_body(x0_ref, x1_ref, qwT_ref, qb_ref, kvwT_ref, q_ref, kvx_ref):
    q_ref[...] = (
        jnp.dot(x0_ref[...], qwT_ref[...], preferred_element_type=_f32)
        + qb_ref[...]
    )
    kvx_ref[...] = jnp.dot(x1_ref[...], kvwT_ref[...], preferred_element_type=_f32)


def _node_tc(x0, x1, qwT, qb, kvwT):
    return pl.pallas_call(
        _node_body,
        out_shape=(
            jax.ShapeDtypeStruct((N, HID), _f32),
            jax.ShapeDtypeStruct((N, 2 * HID), _f32),
        ),
    )(x0, x1, qwT, qb, kvwT)


@functools.partial(
    pl.kernel,
    out_type=(
        jax.ShapeDtypeStruct((E, HID), _f32),
        jax.ShapeDtypeStruct((E, 2 * HID), _f32),
    ),
    mesh=_mesh,
    scratch_types=[
        pltpu.VMEM((CH,), jnp.int32),
        pltpu.VMEM((CH,), jnp.int32),
        pltpu.VMEM((CH, HID), _f32),
        pltpu.VMEM((CH, 2 * HID), _f32),
        pltpu.SemaphoreType.DMA,
        pltpu.SemaphoreType.DMA,
    ],
)
def _gather_sc(q_hbm, kvx_hbm, dst_hbm, src_hbm, qg_hbm, kvxg_hbm,
               di_v, si_v, qrow_v, kvrow_v, sem1, sem2):
    wid = lax.axis_index("s") * NC + lax.axis_index("c")
    base = wid * PER_W

    @pl.loop(0, NCHUNK)
    def _(ci):
        off = base + ci * CH
        pltpu.sync_copy(dst_hbm.at[pl.ds(off, CH)], di_v)
        pltpu.sync_copy(src_hbm.at[pl.ds(off, CH)], si_v)
        c1 = pltpu.async_copy(q_hbm.at[di_v], qrow_v, sem1)
        c2 = pltpu.async_copy(kvx_hbm.at[si_v], kvrow_v, sem2)
        c1.wait()
        c2.wait()
        pltpu.sync_copy(qrow_v, qg_hbm.at[pl.ds(off, CH)])
        pltpu.sync_copy(kvrow_v, kvxg_hbm.at[pl.ds(off, CH)])


BE = 512
GE = E // BE


def _edge_body(ea_ref, qg_ref, kvxg_ref, kwT_ref, kb_ref, vwT_ref, vb_ref,
               msg_ref, ex_ref):
    ea = ea_ref[...]
    k = (
        jnp.dot(ea, kwT_ref[...], preferred_element_type=_f32)
        + kvxg_ref[:, :HID]
        + kb_ref[...]
    )
    s = jnp.sum(k * qg_ref[...], axis=1) * (1.0 / math.sqrt(HID))
    s = jnp.where(s >= 0.0, s, 0.2 * s)
    ex = jnp.exp(s)
    v = (
        jnp.dot(ea, vwT_ref[...], preferred_element_type=_f32)
        + kvxg_ref[:, HID:]
        + vb_ref[...]
    )
    msg_ref[...] = v * ex[:, None]
    ex_ref[...] = ex


def _edge_tc(ea, qg, kvxg, kwT, kb, vwT, vb):
    return pl.pallas_call(
        _edge_body,
        grid=(GE,),
        in_specs=[
            pl.BlockSpec((BE, HID), lambda i: (i, 0)),
            pl.BlockSpec((BE, HID), lambda i: (i, 0)),
            pl.BlockSpec((BE, 2 * HID), lambda i: (i, 0)),
            pl.BlockSpec((HID, HID), lambda i: (0, 0)),
            pl.BlockSpec((1, HID), lambda i: (0, 0)),
            pl.BlockSpec((HID, HID), lambda i: (0, 0)),
            pl.BlockSpec((1, HID), lambda i: (0, 0)),
        ],
        out_specs=(
            pl.BlockSpec((BE, HID), lambda i: (i, 0)),
            pl.BlockSpec((BE,), lambda i: (i,)),
        ),
        out_shape=(
            jax.ShapeDtypeStruct((E, HID), _f32),
            jax.ShapeDtypeStruct((E,), _f32),
        ),
    )(ea, qg, kvxg, kwT, kb, vwT, vb)


@functools.partial(
    pl.kernel,
    out_type=jax.ShapeDtypeStruct((NC * N_PAD, HID), _f32),
    mesh=_mesh,
    scratch_types=[
        pltpu.VMEM((CH, HID), _f32),
        pltpu.VMEM((CH,), jnp.int32),
        pltpu.VMEM_SHARED((N_PAD, HID), _f32),
    ],
    compiler_params=_sc_params,
)
def _scatter_sc(msg_hbm, dst_hbm, z128_hbm, numer_hbm, msg_v, di_v, acc_sh):
    cid = lax.axis_index("c")
    sid = lax.axis_index("s")
    wid = sid * NC + cid
    rbase = sid * RSUB

    pltpu.sync_copy(z128_hbm.at[pl.ds(rbase, RSUB)],
                    acc_sh.at[pl.ds(rbase, RSUB)])
    plsc.subcore_barrier()

    base = wid * PER_W

    @pl.loop(0, NCHUNK)
    def _(ci):
        off = base + ci * CH
        pltpu.sync_copy(msg_hbm.at[pl.ds(off, CH)], msg_v)
        pltpu.sync_copy(dst_hbm.at[pl.ds(off, CH)], di_v)
        pltpu.sync_copy(msg_v, acc_sh.at[di_v], add=True)

    plsc.subcore_barrier()
    pltpu.sync_copy(acc_sh.at[pl.ds(rbase, RSUB)],
                    numer_hbm.at[pl.ds(cid * N_PAD + rbase, RSUB)])


@functools.partial(
    pl.kernel,
    out_type=jax.ShapeDtypeStruct((NC * N_PAD, HID), _f32),
    mesh=_mesh,
    scratch_types=[
        pltpu.VMEM((CH,), _f32),
        pltpu.VMEM((CH, HID), _f32),
        pltpu.VMEM((CH,), jnp.int32),
        pltpu.VMEM_SHARED((N_PAD, HID), _f32),
    ],
    compiler_params=_sc_params,
)
def _denom_sc(ex_hbm, dst_hbm, z128_hbm, denom_hbm, ex_v, exw_v, di_v, den_sh):
    cid = lax.axis_index("c")
    sid = lax.axis_index("s")
    wid = sid * NC + cid
    rbase = sid * RSUB

    pltpu.sync_copy(z128_hbm.at[pl.ds(rbase, RSUB)],
                    den_sh.at[pl.ds(rbase, RSUB)])

    @pl.loop(0, CH)
    def _(r):
        for c in range(0, HID, 16):
            exw_v[r, pl.ds(c, 16)] = jnp.zeros((16,), _f32)

    plsc.subcore_barrier()

    base = wid * PER_W
    lane = lax.iota(jnp.int32, 16)

    @pl.loop(0, NCHUNK)
    def _(ci):
        off = base + ci * CH
        pltpu.sync_copy(ex_hbm.at[pl.ds(off, CH)], ex_v)
        pltpu.sync_copy(dst_hbm.at[pl.ds(off, CH)], di_v)

        @pl.loop(0, CH, step=16)
        def _(i):
            ex16 = ex_v[pl.ds(i, 16)]
            for j in range(16):
                exw_v[i + j, pl.ds(0, 16)] = jnp.where(lane == 0, ex16[j], 0.0)

        pltpu.sync_copy(exw_v, den_sh.at[di_v], add=True)

    plsc.subcore_barrier()
    pltpu.sync_copy(den_sh.at[pl.ds(rbase, RSUB)],
                    denom_hbm.at[pl.ds(cid * N_PAD + rbase, RSUB)])


def _final_body(np_ref, dp_ref, x0_ref, wwT_ref, wb_ref, wihT_ref, whhT_ref,
                bih_ref, bhh_ref, out_ref):
    numer = np_ref[:N] + np_ref[N_PAD:N_PAD + N]
    den = dp_ref[:N, 0:1] + dp_ref[N_PAD:N_PAD + N, 0:1]
    aggr = numer / (den + 1e-16)
    attn = (
        jnp.dot(aggr, wwT_ref[...], preferred_element_type=_f32) + wb_ref[...]
    )
    gi = (
        jnp.dot(x0_ref[...], wihT_ref[...], preferred_element_type=_f32)
        + bih_ref[...]
    )
    gh = (
        jnp.dot(attn, whhT_ref[...], preferred_element_type=_f32)
        + bhh_ref[...]
    )
    r = jax.nn.sigmoid(gi[:, :HID] + gh[:, :HID])
    z = jax.nn.sigmoid(gi[:, HID:2 * HID] + gh[:, HID:2 * HID])
    n = jnp.tanh(gi[:, 2 * HID:] + r * gh[:, 2 * HID:])
    out_ref[...] = (1.0 - z) * n + z * attn


def _final_tc(numer, denom, x0, wwT, wb, wihT, whhT, bih, bhh):
    return pl.pallas_call(
        _final_body,
        out_shape=jax.ShapeDtypeStruct((N, HID), _f32),
    )(numer, denom, x0, wwT, wb, wihT, whhT, bih, bhh)


def kernel(x_0, x_1, edges, edge_types, edge_attr, Qw, Qb, Kw, Kb, Vw, Vb,
           Ww, Wb, w_ih, w_hh, b_ih, b_hh):
    src = edges[0].astype(jnp.int32)
    dst = edges[1].astype(jnp.int32)
    kvwT = jnp.concatenate([Kw[:, HID:].T, Vw[:, HID:].T], axis=1)

    q, kvx = _node_tc(x_0, x_1, Qw.T, Qb.reshape(1, HID), kvwT)
    qg, kvxg = _gather_sc(q, kvx, dst, src)
    msg, ex = _edge_tc(edge_attr, qg, kvxg,
                       Kw[:, :HID].T, Kb.reshape(1, HID),
                       Vw[:, :HID].T, Vb.reshape(1, HID))
    z128 = jnp.zeros((N_PAD, HID), _f32)
    numer = _scatter_sc(msg, dst, z128)
    denom = _denom_sc(ex, dst, z128)
    return _final_tc(numer, denom, x_0, Ww.T, Wb.reshape(1, HID),
                     w_ih.T, w_hh.T, b_ih.reshape(1, 3 * HID),
                     b_hh.reshape(1, 3 * HID))


# speedup vs baseline: 5.1645x; 1.1045x over previous
import dataclasses
import functools
import math

import jax
import jax.numpy as jnp
from jax import lax
from jax.experimental import pallas as pl
from jax.experimental.pallas import tpu as pltpu
from jax.experimental.pallas import tpu_sc as plsc

N = 10000
E = 320000
HID = 128

NC = 2
NS = 16
NW = NC * NS
PER_W = E // NW
CH = 80
NCHUNK = PER_W // CH
N_PAD = 10240
RSUB = N_PAD // NS

_f32 = jnp.float32
_mesh = plsc.VectorSubcoreMesh(core_axis_name="c", subcore_axis_name="s")

_sc_params = pltpu.CompilerParams()
if "needs_layout_passes" in pltpu.CompilerParams.__dataclass_fields__:
    _sc_params = dataclasses.replace(_sc_params, needs_layout_passes=False)


def _node_body(x0_ref, x1_ref, qwT_ref, qb_ref, kvwT_ref, q_ref, kvx_ref):
    q_ref[...] = (
        jnp.dot(x0_ref[...], qwT_ref[...], preferred_element_type=_f32)
        + qb_ref[...]
    )
    kvx_ref[...] = jnp.dot(x1_ref[...], kvwT_ref[...], preferred_element_type=_f32)


def _node_tc(x0, x1, qwT, qb, kvwT):
    return pl.pallas_call(
        _node_body,
        out_shape=(
            jax.ShapeDtypeStruct((N, HID), _f32),
            jax.ShapeDtypeStruct((N, 2 * HID), _f32),
        ),
    )(x0, x1, qwT, qb, kvwT)


GCH = 128
GFULL = PER_W // GCH
GTAIL = PER_W - GFULL * GCH


@functools.partial(
    pl.kernel,
    out_type=(
        jax.ShapeDtypeStruct((E, HID), _f32),
        jax.ShapeDtypeStruct((E, 2 * HID), _f32),
    ),
    mesh=_mesh,
    scratch_types=[
        pltpu.VMEM((GCH,), jnp.int32),
        pltpu.VMEM((GCH,), jnp.int32),
        pltpu.VMEM((GCH,), jnp.int32),
        pltpu.VMEM((GCH,), jnp.int32),
        pltpu.VMEM((GCH, HID), _f32),
        pltpu.VMEM((GCH, HID), _f32),
        pltpu.VMEM((GCH, 2 * HID), _f32),
        pltpu.VMEM((GCH, 2 * HID), _f32),
        pltpu.SemaphoreType.DMA,
        pltpu.SemaphoreType.DMA,
        pltpu.SemaphoreType.DMA,
        pltpu.SemaphoreType.DMA,
        pltpu.SemaphoreType.DMA,
        pltpu.SemaphoreType.DMA,
    ],
)
def _gather_sc(q_hbm, kvx_hbm, dst_hbm, src_hbm, qg_hbm, kvxg_hbm,
               di0, si0, di1, si1, q0, q1, kv0, kv1,
               sI0, sI1, sG0, sG1, sW0, sW1):
    wid = lax.axis_index("s") * NC + lax.axis_index("c")
    base = wid * PER_W

    @pl.loop(0, GFULL, step=2)
    def _(g):
        off0 = base + g * GCH
        off1 = off0 + GCH
        d0 = pltpu.async_copy(dst_hbm.at[pl.ds(off0, GCH)], di0, sI0)
        d1 = pltpu.async_copy(src_hbm.at[pl.ds(off0, GCH)], si0, sI0)
        d2 = pltpu.async_copy(dst_hbm.at[pl.ds(off1, GCH)], di1, sI1)
        d3 = pltpu.async_copy(src_hbm.at[pl.ds(off1, GCH)], si1, sI1)
        d0.wait()
        d1.wait()
        g0 = pltpu.async_copy(q_hbm.at[di0], q0, sG0)
        g1 = pltpu.async_copy(kvx_hbm.at[si0], kv0, sG0)
        d2.wait()
        d3.wait()
        g2 = pltpu.async_copy(q_hbm.at[di1], q1, sG1)
        g3 = pltpu.async_copy(kvx_hbm.at[si1], kv1, sG1)
        g0.wait()
        g1.wait()
        w0 = pltpu.async_copy(q0, qg_hbm.at[pl.ds(off0, GCH)], sW0)
        w1 = pltpu.async_copy(kv0, kvxg_hbm.at[pl.ds(off0, GCH)], sW0)
        g2.wait()
        g3.wait()
        w2 = pltpu.async_copy(q1, qg_hbm.at[pl.ds(off1, GCH)], sW1)
        w3 = pltpu.async_copy(kv1, kvxg_hbm.at[pl.ds(off1, GCH)], sW1)
        w0.wait()
        w1.wait()
        w2.wait()
        w3.wait()

    toff = base + GFULL * GCH
    pltpu.sync_copy(dst_hbm.at[pl.ds(toff, GTAIL)], di0.at[pl.ds(0, GTAIL)])
    pltpu.sync_copy(src_hbm.at[pl.ds(toff, GTAIL)], si0.at[pl.ds(0, GTAIL)])
    t0 = pltpu.async_copy(q_hbm.at[di0.at[pl.ds(0, GTAIL)]],
                          q0.at[pl.ds(0, GTAIL)], sG0)
    t1 = pltpu.async_copy(kvx_hbm.at[si0.at[pl.ds(0, GTAIL)]],
                          kv0.at[pl.ds(0, GTAIL)], sG1)
    t0.wait()
    t1.wait()
    pltpu.sync_copy(q0.at[pl.ds(0, GTAIL)], qg_hbm.at[pl.ds(toff, GTAIL)])
    pltpu.sync_copy(kv0.at[pl.ds(0, GTAIL)], kvxg_hbm.at[pl.ds(toff, GTAIL)])


BE = 512
GE = E // BE


def _edge_body(ea_ref, qg_ref, kvxg_ref, kwT_ref, kb_ref, vwT_ref, vb_ref,
               msg_ref, ex_ref):
    ea = ea_ref[...]
    k = (
        jnp.dot(ea, kwT_ref[...], preferred_element_type=_f32)
        + kvxg_ref[:, :HID]
        + kb_ref[...]
    )
    s = jnp.sum(k * qg_ref[...], axis=1) * (1.0 / math.sqrt(HID))
    s = jnp.where(s >= 0.0, s, 0.2 * s)
    ex = jnp.exp(s)
    v = (
        jnp.dot(ea, vwT_ref[...], preferred_element_type=_f32)
        + kvxg_ref[:, HID:]
        + vb_ref[...]
    )
    msg_ref[...] = v * ex[:, None]
    ex_ref[...] = ex


def _edge_tc(ea, qg, kvxg, kwT, kb, vwT, vb):
    return pl.pallas_call(
        _edge_body,
        grid=(GE,),
        in_specs=[
            pl.BlockSpec((BE, HID), lambda i: (i, 0)),
            pl.BlockSpec((BE, HID), lambda i: (i, 0)),
            pl.BlockSpec((BE, 2 * HID), lambda i: (i, 0)),
            pl.BlockSpec((HID, HID), lambda i: (0, 0)),
            pl.BlockSpec((1, HID), lambda i: (0, 0)),
            pl.BlockSpec((HID, HID), lambda i: (0, 0)),
            pl.BlockSpec((1, HID), lambda i: (0, 0)),
        ],
        out_specs=(
            pl.BlockSpec((BE, HID), lambda i: (i, 0)),
            pl.BlockSpec((BE,), lambda i: (i,)),
        ),
        out_shape=(
            jax.ShapeDtypeStruct((E, HID), _f32),
            jax.ShapeDtypeStruct((E,), _f32),
        ),
    )(ea, qg, kvxg, kwT, kb, vwT, vb)


@functools.partial(
    pl.kernel,
    out_type=jax.ShapeDtypeStruct((NC * N_PAD, HID), _f32),
    mesh=_mesh,
    scratch_types=[
        pltpu.VMEM((CH, HID), _f32),
        pltpu.VMEM((CH,), jnp.int32),
        pltpu.VMEM_SHARED((N_PAD, HID), _f32),
    ],
    compiler_params=_sc_params,
)
def _scatter_sc(msg_hbm, dst_hbm, z128_hbm, numer_hbm, msg_v, di_v, acc_sh):
    cid = lax.axis_index("c")
    sid = lax.axis_index("s")
    wid = sid * NC + cid
    rbase = sid * RSUB

    pltpu.sync_copy(z128_hbm.at[pl.ds(rbase, RSUB)],
                    acc_sh.at[pl.ds(rbase, RSUB)])
    plsc.subcore_barrier()

    base = wid * PER_W

    @pl.loop(0, NCHUNK)
    def _(ci):
        off = base + ci * CH
        pltpu.sync_copy(msg_hbm.at[pl.ds(off, CH)], msg_v)
        pltpu.sync_copy(dst_hbm.at[pl.ds(off, CH)], di_v)
        pltpu.sync_copy(msg_v, acc_sh.at[di_v], add=True)

    plsc.subcore_barrier()
    pltpu.sync_copy(acc_sh.at[pl.ds(rbase, RSUB)],
                    numer_hbm.at[pl.ds(cid * N_PAD + rbase, RSUB)])


@functools.partial(
    pl.kernel,
    out_type=jax.ShapeDtypeStruct((NC * N_PAD, HID), _f32),
    mesh=_mesh,
    scratch_types=[
        pltpu.VMEM((CH,), _f32),
        pltpu.VMEM((CH, HID), _f32),
        pltpu.VMEM((CH,), jnp.int32),
        pltpu.VMEM_SHARED((N_PAD, HID), _f32),
    ],
    compiler_params=_sc_params,
)
def _denom_sc(ex_hbm, dst_hbm, z128_hbm, denom_hbm, ex_v, exw_v, di_v, den_sh):
    cid = lax.axis_index("c")
    sid = lax.axis_index("s")
    wid = sid * NC + cid
    rbase = sid * RSUB

    pltpu.sync_copy(z128_hbm.at[pl.ds(rbase, RSUB)],
                    den_sh.at[pl.ds(rbase, RSUB)])

    @pl.loop(0, CH)
    def _(r):
        for c in range(0, HID, 16):
            exw_v[r, pl.ds(c, 16)] = jnp.zeros((16,), _f32)

    plsc.subcore_barrier()

    base = wid * PER_W
    lane = lax.iota(jnp.int32, 16)

    @pl.loop(0, NCHUNK)
    def _(ci):
        off = base + ci * CH
        pltpu.sync_copy(ex_hbm.at[pl.ds(off, CH)], ex_v)
        pltpu.sync_copy(dst_hbm.at[pl.ds(off, CH)], di_v)

        @pl.loop(0, CH, step=16)
        def _(i):
            ex16 = ex_v[pl.ds(i, 16)]
            for j in range(16):
                exw_v[i + j, pl.ds(0, 16)] = jnp.where(lane == 0, ex16[j], 0.0)

        pltpu.sync_copy(exw_v, den_sh.at[di_v], add=True)

    plsc.subcore_barrier()
    pltpu.sync_copy(den_sh.at[pl.ds(rbase, RSUB)],
                    denom_hbm.at[pl.ds(cid * N_PAD + rbase, RSUB)])


def _final_body(np_ref, dp_ref, x0_ref, wwT_ref, wb_ref, wihT_ref, whhT_ref,
                bih_ref, bhh_ref, out_ref):
    numer = np_ref[:N] + np_ref[N_PAD:N_PAD + N]
    den = dp_ref[:N, 0:1] + dp_ref[N_PAD:N_PAD + N, 0:1]
    aggr = numer / (den + 1e-16)
    attn = (
        jnp.dot(aggr, wwT_ref[...], preferred_element_type=_f32) + wb_ref[...]
    )
    gi = (
        jnp.dot(x0_ref[...], wihT_ref[...], preferred_element_type=_f32)
        + bih_ref[...]
    )
    gh = (
        jnp.dot(attn, whhT_ref[...], preferred_element_type=_f32)
        + bhh_ref[...]
    )
    r = jax.nn.sigmoid(gi[:, :HID] + gh[:, :HID])
    z = jax.nn.sigmoid(gi[:, HID:2 * HID] + gh[:, HID:2 * HID])
    n = jnp.tanh(gi[:, 2 * HID:] + r * gh[:, 2 * HID:])
    out_ref[...] = (1.0 - z) * n + z * attn


def _final_tc(numer, denom, x0, wwT, wb, wihT, whhT, bih, bhh):
    return pl.pallas_call(
        _final_body,
        out_shape=jax.ShapeDtypeStruct((N, HID), _f32),
    )(numer, denom, x0, wwT, wb, wihT, whhT, bih, bhh)


def kernel(x_0, x_1, edges, edge_types, edge_attr, Qw, Qb, Kw, Kb, Vw, Vb,
           Ww, Wb, w_ih, w_hh, b_ih, b_hh):
    src = edges[0].astype(jnp.int32)
    dst = edges[1].astype(jnp.int32)
    kvwT = jnp.concatenate([Kw[:, HID:].T, Vw[:, HID:].T], axis=1)

    q, kvx = _node_tc(x_0, x_1, Qw.T, Qb.reshape(1, HID), kvwT)
    qg, kvxg = _gather_sc(q, kvx, dst, src)
    msg, ex = _edge_tc(edge_attr, qg, kvxg,
                       Kw[:, :HID].T, Kb.reshape(1, HID),
                       Vw[:, :HID].T, Vb.reshape(1, HID))
    z128 = jnp.zeros((N_PAD, HID), _f32)
    numer = _scatter_sc(msg, dst, z128)
    denom = _denom_sc(ex, dst, z128)
    return _final_tc(numer, denom, x_0, Ww.T, Wb.reshape(1, HID),
                     w_ih.T, w_hh.T, b_ih.reshape(1, 3 * HID),
                     b_hh.reshape(1, 3 * HID))


# speedup vs baseline: 5.9347x; 1.1491x over previous
import dataclasses
import functools
import math

import jax
import jax.numpy as jnp
from jax import lax
from jax.experimental import pallas as pl
from jax.experimental.pallas import tpu as pltpu
from jax.experimental.pallas import tpu_sc as plsc

N = 10000
E = 320000
HID = 128

NC = 2
NS = 16
NW = NC * NS
PER_W = E // NW
CH = 80
NCHUNK = PER_W // CH
N_PAD = 10240
RSUB = N_PAD // NS

_f32 = jnp.float32
_mesh = plsc.VectorSubcoreMesh(core_axis_name="c", subcore_axis_name="s")

_sc_params = pltpu.CompilerParams()
if "needs_layout_passes" in pltpu.CompilerParams.__dataclass_fields__:
    _sc_params = dataclasses.replace(_sc_params, needs_layout_passes=False)


def _node_body(x0_ref, x1_ref, qwT_ref, qb_ref, kvwT_ref, q_ref, kvx_ref):
    q_ref[...] = (
        jnp.dot(x0_ref[...], qwT_ref[...], preferred_element_type=_f32)
        + qb_ref[...]
    )
    kvx_ref[...] = jnp.dot(x1_ref[...], kvwT_ref[...], preferred_element_type=_f32)


def _node_tc(x0, x1, qwT, qb, kvwT):
    return pl.pallas_call(
        _node_body,
        out_shape=(
            jax.ShapeDtypeStruct((N, HID), _f32),
            jax.ShapeDtypeStruct((N, 2 * HID), _f32),
        ),
    )(x0, x1, qwT, qb, kvwT)


GCH = 128
GFULL = PER_W // GCH
GTAIL = PER_W - GFULL * GCH


@functools.partial(
    pl.kernel,
    out_type=(
        jax.ShapeDtypeStruct((E, HID), _f32),
        jax.ShapeDtypeStruct((E, 2 * HID), _f32),
    ),
    mesh=_mesh,
    scratch_types=[
        pltpu.VMEM((GCH,), jnp.int32),
        pltpu.VMEM((GCH,), jnp.int32),
        pltpu.VMEM((GCH,), jnp.int32),
        pltpu.VMEM((GCH,), jnp.int32),
        pltpu.VMEM((GCH, HID), _f32),
        pltpu.VMEM((GCH, HID), _f32),
        pltpu.VMEM((GCH, 2 * HID), _f32),
        pltpu.VMEM((GCH, 2 * HID), _f32),
        pltpu.SemaphoreType.DMA,
        pltpu.SemaphoreType.DMA,
        pltpu.SemaphoreType.DMA,
        pltpu.SemaphoreType.DMA,
        pltpu.SemaphoreType.DMA,
        pltpu.SemaphoreType.DMA,
    ],
)
def _gather_sc(q_hbm, kvx_hbm, dst_hbm, src_hbm, qg_hbm, kvxg_hbm,
               di0, si0, di1, si1, q0, q1, kv0, kv1,
               sI0, sI1, sG0, sG1, sW0, sW1):
    wid = lax.axis_index("s") * NC + lax.axis_index("c")
    base = wid * PER_W

    @pl.loop(0, GFULL, step=2)
    def _(g):
        off0 = base + g * GCH
        off1 = off0 + GCH
        d0 = pltpu.async_copy(dst_hbm.at[pl.ds(off0, GCH)], di0, sI0)
        d1 = pltpu.async_copy(src_hbm.at[pl.ds(off0, GCH)], si0, sI0)
        d2 = pltpu.async_copy(dst_hbm.at[pl.ds(off1, GCH)], di1, sI1)
        d3 = pltpu.async_copy(src_hbm.at[pl.ds(off1, GCH)], si1, sI1)
        d0.wait()
        d1.wait()
        g0 = pltpu.async_copy(q_hbm.at[di0], q0, sG0)
        g1 = pltpu.async_copy(kvx_hbm.at[si0], kv0, sG0)
        d2.wait()
        d3.wait()
        g2 = pltpu.async_copy(q_hbm.at[di1], q1, sG1)
        g3 = pltpu.async_copy(kvx_hbm.at[si1], kv1, sG1)
        g0.wait()
        g1.wait()
        w0 = pltpu.async_copy(q0, qg_hbm.at[pl.ds(off0, GCH)], sW0)
        w1 = pltpu.async_copy(kv0, kvxg_hbm.at[pl.ds(off0, GCH)], sW0)
        g2.wait()
        g3.wait()
        w2 = pltpu.async_copy(q1, qg_hbm.at[pl.ds(off1, GCH)], sW1)
        w3 = pltpu.async_copy(kv1, kvxg_hbm.at[pl.ds(off1, GCH)], sW1)
        w0.wait()
        w1.wait()
        w2.wait()
        w3.wait()

    toff = base + GFULL * GCH
    pltpu.sync_copy(dst_hbm.at[pl.ds(toff, GTAIL)], di0.at[pl.ds(0, GTAIL)])
    pltpu.sync_copy(src_hbm.at[pl.ds(toff, GTAIL)], si0.at[pl.ds(0, GTAIL)])
    t0 = pltpu.async_copy(q_hbm.at[di0.at[pl.ds(0, GTAIL)]],
                          q0.at[pl.ds(0, GTAIL)], sG0)
    t1 = pltpu.async_copy(kvx_hbm.at[si0.at[pl.ds(0, GTAIL)]],
                          kv0.at[pl.ds(0, GTAIL)], sG1)
    t0.wait()
    t1.wait()
    pltpu.sync_copy(q0.at[pl.ds(0, GTAIL)], qg_hbm.at[pl.ds(toff, GTAIL)])
    pltpu.sync_copy(kv0.at[pl.ds(0, GTAIL)], kvxg_hbm.at[pl.ds(toff, GTAIL)])


BE = 512
GE = E // BE


def _edge_body(ea_ref, qg_ref, kvxg_ref, kwT_ref, kb_ref, vwT_ref, vb_ref,
               msg_ref, ex_ref):
    ea = ea_ref[...]
    k = (
        jnp.dot(ea, kwT_ref[...], preferred_element_type=_f32)
        + kvxg_ref[:, :HID]
        + kb_ref[...]
    )
    s = jnp.sum(k * qg_ref[...], axis=1) * (1.0 / math.sqrt(HID))
    s = jnp.where(s >= 0.0, s, 0.2 * s)
    ex = jnp.exp(s)
    v = (
        jnp.dot(ea, vwT_ref[...], preferred_element_type=_f32)
        + kvxg_ref[:, HID:]
        + vb_ref[...]
    )
    msg_ref[...] = v * ex[:, None]
    ex_ref[...] = ex


def _edge_tc(ea, qg, kvxg, kwT, kb, vwT, vb):
    return pl.pallas_call(
        _edge_body,
        grid=(GE,),
        in_specs=[
            pl.BlockSpec((BE, HID), lambda i: (i, 0)),
            pl.BlockSpec((BE, HID), lambda i: (i, 0)),
            pl.BlockSpec((BE, 2 * HID), lambda i: (i, 0)),
            pl.BlockSpec((HID, HID), lambda i: (0, 0)),
            pl.BlockSpec((1, HID), lambda i: (0, 0)),
            pl.BlockSpec((HID, HID), lambda i: (0, 0)),
            pl.BlockSpec((1, HID), lambda i: (0, 0)),
        ],
        out_specs=(
            pl.BlockSpec((BE, HID), lambda i: (i, 0)),
            pl.BlockSpec((BE,), lambda i: (i,)),
        ),
        out_shape=(
            jax.ShapeDtypeStruct((E, HID), _f32),
            jax.ShapeDtypeStruct((E,), _f32),
        ),
    )(ea, qg, kvxg, kwT, kb, vwT, vb)


@functools.partial(
    pl.kernel,
    out_type=jax.ShapeDtypeStruct((NC * N_PAD, HID), _f32),
    mesh=_mesh,
    scratch_types=[
        pltpu.VMEM((GCH, HID), _f32),
        pltpu.VMEM((GCH, HID), _f32),
        pltpu.VMEM((GCH,), jnp.int32),
        pltpu.VMEM((GCH,), jnp.int32),
        pltpu.VMEM((GTAIL,), jnp.int32),
        pltpu.VMEM_SHARED((N_PAD, HID), _f32),
        pltpu.SemaphoreType.DMA,
        pltpu.SemaphoreType.DMA,
        pltpu.SemaphoreType.DMA,
        pltpu.SemaphoreType.DMA,
    ],
    compiler_params=_sc_params,
)
def _scatter_sc(msg_hbm, dst_hbm, z128_hbm, numer_hbm,
                msg0, msg1, di0, di1, dit, acc_sh, sL0, sL1, sA0, sA1):
    cid = lax.axis_index("c")
    sid = lax.axis_index("s")
    wid = sid * NC + cid
    rbase = sid * RSUB

    pltpu.sync_copy(z128_hbm.at[pl.ds(rbase, RSUB)],
                    acc_sh.at[pl.ds(rbase, RSUB)])
    plsc.subcore_barrier()

    base = wid * PER_W

    @pl.loop(0, GFULL, step=2)
    def _(g):
        off0 = base + g * GCH
        off1 = off0 + GCH
        m0 = pltpu.async_copy(msg_hbm.at[pl.ds(off0, GCH)], msg0, sL0)
        i0 = pltpu.async_copy(dst_hbm.at[pl.ds(off0, GCH)], di0, sL0)
        m1 = pltpu.async_copy(msg_hbm.at[pl.ds(off1, GCH)], msg1, sL1)
        i1 = pltpu.async_copy(dst_hbm.at[pl.ds(off1, GCH)], di1, sL1)
        m0.wait()
        i0.wait()
        a0 = pltpu.async_copy(msg0, acc_sh.at[di0], sA0, add=True)
        m1.wait()
        i1.wait()
        a1 = pltpu.async_copy(msg1, acc_sh.at[di1], sA1, add=True)
        a0.wait()
        a1.wait()

    toff = base + GFULL * GCH
    pltpu.sync_copy(msg_hbm.at[pl.ds(toff, GTAIL)], msg0.at[pl.ds(0, GTAIL)])
    pltpu.sync_copy(dst_hbm.at[pl.ds(toff, GTAIL)], dit)
    pltpu.sync_copy(msg0.at[pl.ds(0, GTAIL)], acc_sh.at[dit], add=True)

    plsc.subcore_barrier()
    pltpu.sync_copy(acc_sh.at[pl.ds(rbase, RSUB)],
                    numer_hbm.at[pl.ds(cid * N_PAD + rbase, RSUB)])


@functools.partial(
    pl.kernel,
    out_type=jax.ShapeDtypeStruct((NC * N_PAD, HID), _f32),
    mesh=_mesh,
    scratch_types=[
        pltpu.VMEM((GCH,), _f32),
        pltpu.VMEM((GCH,), _f32),
        pltpu.VMEM((GCH, HID), _f32),
        pltpu.VMEM((GCH, HID), _f32),
        pltpu.VMEM((GCH,), jnp.int32),
        pltpu.VMEM((GCH,), jnp.int32),
        pltpu.VMEM((GTAIL,), jnp.int32),
        pltpu.VMEM_SHARED((N_PAD, HID), _f32),
        pltpu.SemaphoreType.DMA,
        pltpu.SemaphoreType.DMA,
        pltpu.SemaphoreType.DMA,
        pltpu.SemaphoreType.DMA,
    ],
    compiler_params=_sc_params,
)
def _denom_sc(ex_hbm, dst_hbm, z128_hbm, denom_hbm,
              ex0, ex1, exw0, exw1, di0, di1, dit, den_sh,
              sL0, sL1, sA0, sA1):
    cid = lax.axis_index("c")
    sid = lax.axis_index("s")
    wid = sid * NC + cid
    rbase = sid * RSUB

    pltpu.sync_copy(z128_hbm.at[pl.ds(rbase, RSUB)],
                    den_sh.at[pl.ds(rbase, RSUB)])

    @pl.loop(0, GCH)
    def _(r):
        for c in range(0, HID, 16):
            exw0[r, pl.ds(c, 16)] = jnp.zeros((16,), _f32)
            exw1[r, pl.ds(c, 16)] = jnp.zeros((16,), _f32)

    plsc.subcore_barrier()

    base = wid * PER_W
    lane = lax.iota(jnp.int32, 16)

    def _stage(exv, exw):
        @pl.loop(0, GCH, step=16)
        def _(i):
            ex16 = exv[pl.ds(i, 16)]
            for j in range(16):
                exw[i + j, pl.ds(0, 16)] = jnp.where(lane == 0, ex16[j], 0.0)

    @pl.loop(0, GFULL, step=2)
    def _(g):
        off0 = base + g * GCH
        off1 = off0 + GCH
        e0 = pltpu.async_copy(ex_hbm.at[pl.ds(off0, GCH)], ex0, sL0)
        i0 = pltpu.async_copy(dst_hbm.at[pl.ds(off0, GCH)], di0, sL0)
        e1 = pltpu.async_copy(ex_hbm.at[pl.ds(off1, GCH)], ex1, sL1)
        i1 = pltpu.async_copy(dst_hbm.at[pl.ds(off1, GCH)], di1, sL1)
        e0.wait()
        i0.wait()
        _stage(ex0, exw0)
        a0 = pltpu.async_copy(exw0, den_sh.at[di0], sA0, add=True)
        e1.wait()
        i1.wait()
        _stage(ex1, exw1)
        a1 = pltpu.async_copy(exw1, den_sh.at[di1], sA1, add=True)
        a0.wait()
        a1.wait()

    toff = base + GFULL * GCH
    pltpu.sync_copy(ex_hbm.at[pl.ds(toff, GTAIL)], ex0.at[pl.ds(0, GTAIL)])
    pltpu.sync_copy(dst_hbm.at[pl.ds(toff, GTAIL)], dit)
    ext = ex0[pl.ds(0, 16)]
    for j in range(GTAIL):
        exw0[j, pl.ds(0, 16)] = jnp.where(lane == 0, ext[j], 0.0)
    pltpu.sync_copy(exw0.at[pl.ds(0, GTAIL)], den_sh.at[dit], add=True)

    plsc.subcore_barrier()
    pltpu.sync_copy(den_sh.at[pl.ds(rbase, RSUB)],
                    denom_hbm.at[pl.ds(cid * N_PAD + rbase, RSUB)])


def _final_body(np_ref, dp_ref, x0_ref, wwT_ref, wb_ref, wihT_ref, whhT_ref,
                bih_ref, bhh_ref, out_ref):
    numer = np_ref[:N] + np_ref[N_PAD:N_PAD + N]
    den = dp_ref[:N, 0:1] + dp_ref[N_PAD:N_PAD + N, 0:1]
    aggr = numer / (den + 1e-16)
    attn = (
        jnp.dot(aggr, wwT_ref[...], preferred_element_type=_f32) + wb_ref[...]
    )
    gi = (
        jnp.dot(x0_ref[...], wihT_ref[...], preferred_element_type=_f32)
        + bih_ref[...]
    )
    gh = (
        jnp.dot(attn, whhT_ref[...], preferred_element_type=_f32)
        + bhh_ref[...]
    )
    r = jax.nn.sigmoid(gi[:, :HID] + gh[:, :HID])
    z = jax.nn.sigmoid(gi[:, HID:2 * HID] + gh[:, HID:2 * HID])
    n = jnp.tanh(gi[:, 2 * HID:] + r * gh[:, 2 * HID:])
    out_ref[...] = (1.0 - z) * n + z * attn


def _final_tc(numer, denom, x0, wwT, wb, wihT, whhT, bih, bhh):
    return pl.pallas_call(
        _final_body,
        out_shape=jax.ShapeDtypeStruct((N, HID), _f32),
    )(numer, denom, x0, wwT, wb, wihT, whhT, bih, bhh)


def kernel(x_0, x_1, edges, edge_types, edge_attr, Qw, Qb, Kw, Kb, Vw, Vb,
           Ww, Wb, w_ih, w_hh, b_ih, b_hh):
    src = edges[0].astype(jnp.int32)
    dst = edges[1].astype(jnp.int32)
    kvwT = jnp.concatenate([Kw[:, HID:].T, Vw[:, HID:].T], axis=1)

    q, kvx = _node_tc(x_0, x_1, Qw.T, Qb.reshape(1, HID), kvwT)
    qg, kvxg = _gather_sc(q, kvx, dst, src)
    msg, ex = _edge_tc(edge_attr, qg, kvxg,
                       Kw[:, :HID].T, Kb.reshape(1, HID),
                       Vw[:, :HID].T, Vb.reshape(1, HID))
    z128 = jnp.zeros((N_PAD, HID), _f32)
    numer = _scatter_sc(msg, dst, z128)
    denom = _denom_sc(ex, dst, z128)
    return _final_tc(numer, denom, x_0, Ww.T, Wb.reshape(1, HID),
                     w_ih.T, w_hh.T, b_ih.reshape(1, 3 * HID),
                     b_hh.reshape(1, 3 * HID))
